# baseline (speedup 1.0000x reference)
import jax
import jax.numpy as jnp
from jax import lax
from jax.experimental import pallas as pl
from jax.experimental.pallas import tpu as pltpu, tpu_sc as plsc

NUM_NODES = 100000
NUM_TYPES = 119
EMB_DIM = 128

NUM_CORES = 2
NUM_SUBCORES = 16
NUM_WORKERS = NUM_CORES * NUM_SUBCORES
CHUNK = 400
NCHUNKS = NUM_NODES // CHUNK
CHUNKS_PER_WORKER = -(-NCHUNKS // NUM_WORKERS)


def _emb_lookup_body(table_hbm, idx_hbm, out_hbm,
                     idx_all, rows0, rows1, sem_i, sem_g, sem_s0, sem_s1):
    wid = lax.axis_index("s") * NUM_CORES + lax.axis_index("c")

    for j in range(CHUNKS_PER_WORKER):
        c = wid + NUM_WORKERS * j

        @pl.when(c < NCHUNKS)
        def _():
            pltpu.async_copy(idx_hbm.at[pl.ds(c * CHUNK, CHUNK)],
                             idx_all.at[pl.ds(j * CHUNK, CHUNK)], sem_i)
    for j in range(CHUNKS_PER_WORKER):
        c = wid + NUM_WORKERS * j

        @pl.when(c < NCHUNKS)
        def _():
            pltpu.make_async_copy(idx_hbm.at[pl.ds(c * CHUNK, CHUNK)],
                                  idx_all.at[pl.ds(j * CHUNK, CHUNK)],
                                  sem_i).wait()

    for j in range(CHUNKS_PER_WORKER):
        c = wid + NUM_WORKERS * j
        rows = rows0 if j % 2 == 0 else rows1
        sem_s = sem_s0 if j % 2 == 0 else sem_s1

        @pl.when(c < NCHUNKS)
        def _():
            if j >= 2:
                pltpu.make_async_copy(
                    rows, out_hbm.at[pl.ds(c * CHUNK, CHUNK)], sem_s).wait()
            pltpu.async_copy(
                table_hbm.at[idx_all.at[pl.ds(j * CHUNK, CHUNK)]],
                rows, sem_g).wait()
            pltpu.async_copy(rows, out_hbm.at[pl.ds(c * CHUNK, CHUNK)], sem_s)

    pltpu.make_async_copy(rows0, out_hbm.at[pl.ds(0, CHUNK)], sem_s0).wait()
    pltpu.make_async_copy(rows1, out_hbm.at[pl.ds(0, CHUNK)], sem_s1).wait()


def kernel(atomic_numbers, emb):
    idx = atomic_numbers.astype(jnp.int32)
    mesh = plsc.VectorSubcoreMesh(
        core_axis_name="c", subcore_axis_name="s",
        num_cores=NUM_CORES, num_subcores=NUM_SUBCORES)
    out = pl.kernel(
        _emb_lookup_body,
        out_type=jax.ShapeDtypeStruct((NUM_NODES, EMB_DIM), jnp.float32),
        mesh=mesh,
        scratch_types=[
            pltpu.VMEM((CHUNK * CHUNKS_PER_WORKER,), jnp.int32),
            pltpu.VMEM((CHUNK, EMB_DIM), jnp.float32),
            pltpu.VMEM((CHUNK, EMB_DIM), jnp.float32),
            pltpu.SemaphoreType.DMA,
            pltpu.SemaphoreType.DMA,
            pltpu.SemaphoreType.DMA,
            pltpu.SemaphoreType.DMA,
        ],
    )(emb, idx)
    return (out, out)


# speedup vs baseline: 1.0285x; 1.0285x over previous
import jax
import jax.numpy as jnp
from jax import lax
from jax.experimental import pallas as pl
from jax.experimental.pallas import tpu as pltpu, tpu_sc as plsc

NUM_NODES = 100000
NUM_TYPES = 119
EMB_DIM = 128

NUM_CORES = 2
NUM_SUBCORES = 16
NUM_WORKERS = NUM_CORES * NUM_SUBCORES
CHUNK = 800
NCHUNKS = NUM_NODES // CHUNK
CHUNKS_PER_WORKER = -(-NCHUNKS // NUM_WORKERS)


def _emb_lookup_body(table_hbm, idx_hbm, out_hbm, idx_v, rows_v, sem):
    wid = lax.axis_index("s") * NUM_CORES + lax.axis_index("c")
    for j in range(CHUNKS_PER_WORKER):
        c = wid + NUM_WORKERS * j

        @pl.when(c < NCHUNKS)
        def _():
            base = c * CHUNK
            pltpu.sync_copy(idx_hbm.at[pl.ds(base, CHUNK)], idx_v)
            pltpu.async_copy(table_hbm.at[idx_v], rows_v, sem).wait()
            pltpu.sync_copy(rows_v, out_hbm.at[pl.ds(base, CHUNK)])


def kernel(atomic_numbers, emb):
    idx = atomic_numbers.astype(jnp.int32)
    mesh = plsc.VectorSubcoreMesh(
        core_axis_name="c", subcore_axis_name="s",
        num_cores=NUM_CORES, num_subcores=NUM_SUBCORES)
    out = pl.kernel(
        _emb_lookup_body,
        out_type=jax.ShapeDtypeStruct((NUM_NODES, EMB_DIM), jnp.float32),
        mesh=mesh,
        scratch_types=[
            pltpu.VMEM((CHUNK,), jnp.int32),
            pltpu.VMEM((CHUNK, EMB_DIM), jnp.float32),
            pltpu.SemaphoreType.DMA,
        ],
    )(emb, idx)
    return (out, out)


# speedup vs baseline: 2.2776x; 2.2144x over previous
import jax
import jax.numpy as jnp
from jax import lax
from jax.experimental import pallas as pl
from jax.experimental.pallas import tpu as pltpu, tpu_sc as plsc

NUM_NODES = 100000
NUM_TYPES = 119
EMB_DIM = 128

NUM_CORES = 2
NUM_SUBCORES = 16
NUM_WORKERS = NUM_CORES * NUM_SUBCORES
CHUNK = 400
NCHUNKS = NUM_NODES // CHUNK
CHUNKS_PER_WORKER = -(-NCHUNKS // NUM_WORKERS)


def _emb_lookup_body(table_hbm, idx_hbm, out_hbm,
                     table_sh, idx_all, rows0, rows1,
                     sem_i, sem_g, sem_s0, sem_s1):
    wid = lax.axis_index("s") * NUM_CORES + lax.axis_index("c")
    sid = lax.axis_index("s")

    @pl.when(sid == 0)
    def _():
        pltpu.sync_copy(table_hbm, table_sh)

    for j in range(CHUNKS_PER_WORKER):
        c = wid + NUM_WORKERS * j

        @pl.when(c < NCHUNKS)
        def _():
            pltpu.async_copy(idx_hbm.at[pl.ds(c * CHUNK, CHUNK)],
                             idx_all.at[pl.ds(j * CHUNK, CHUNK)], sem_i)
    for j in range(CHUNKS_PER_WORKER):
        c = wid + NUM_WORKERS * j

        @pl.when(c < NCHUNKS)
        def _():
            pltpu.make_async_copy(idx_hbm.at[pl.ds(c * CHUNK, CHUNK)],
                                  idx_all.at[pl.ds(j * CHUNK, CHUNK)],
                                  sem_i).wait()

    plsc.subcore_barrier()

    for j in range(CHUNKS_PER_WORKER):
        c = wid + NUM_WORKERS * j
        rows = rows0 if j % 2 == 0 else rows1
        sem_s = sem_s0 if j % 2 == 0 else sem_s1

        @pl.when(c < NCHUNKS)
        def _():
            if j >= 2:
                pltpu.make_async_copy(
                    rows, out_hbm.at[pl.ds(c * CHUNK, CHUNK)], sem_s).wait()
            pltpu.async_copy(
                table_sh.at[idx_all.at[pl.ds(j * CHUNK, CHUNK)]],
                rows, sem_g).wait()
            pltpu.async_copy(rows, out_hbm.at[pl.ds(c * CHUNK, CHUNK)], sem_s)

    pltpu.make_async_copy(rows0, out_hbm.at[pl.ds(0, CHUNK)], sem_s0).wait()
    pltpu.make_async_copy(rows1, out_hbm.at[pl.ds(0, CHUNK)], sem_s1).wait()


def kernel(atomic_numbers, emb):
    idx = atomic_numbers.astype(jnp.int32)
    mesh = plsc.VectorSubcoreMesh(
        core_axis_name="c", subcore_axis_name="s",
        num_cores=NUM_CORES, num_subcores=NUM_SUBCORES)
    out = pl.kernel(
        _emb_lookup_body,
        out_type=jax.ShapeDtypeStruct((NUM_NODES, EMB_DIM), jnp.float32),
        mesh=mesh,
        scratch_types=[
            pltpu.VMEM_SHARED((NUM_TYPES, EMB_DIM), jnp.float32),
            pltpu.VMEM((CHUNK * CHUNKS_PER_WORKER,), jnp.int32),
            pltpu.VMEM((CHUNK, EMB_DIM), jnp.float32),
            pltpu.VMEM((CHUNK, EMB_DIM), jnp.float32),
            pltpu.SemaphoreType.DMA,
            pltpu.SemaphoreType.DMA,
            pltpu.SemaphoreType.DMA,
            pltpu.SemaphoreType.DMA,
        ],
    )(emb, idx)
    return (out, out)


# speedup vs baseline: 3.1364x; 1.3771x over previous
import jax
import jax.numpy as jnp
from jax import lax
from jax.experimental import pallas as pl
from jax.experimental.pallas import tpu as pltpu, tpu_sc as plsc

NUM_NODES = 100000
NUM_TYPES = 119
EMB_DIM = 128

NUM_CORES = 2
NUM_SUBCORES = 16
NUM_WORKERS = NUM_CORES * NUM_SUBCORES
CHUNK = 400
NCHUNKS = NUM_NODES // CHUNK
CHUNKS_PER_WORKER = -(-NCHUNKS // NUM_WORKERS)


def _emb_lookup_body(table_hbm, idx_hbm, out_hbm,
                     table_sh, idx_all, rows0, rows1,
                     sem_i, sem_g, sem_s0, sem_s1):
    wid = lax.axis_index("s") * NUM_CORES + lax.axis_index("c")
    sid = lax.axis_index("s")

    @pl.when(sid == 0)
    def _():
        pltpu.sync_copy(table_hbm, table_sh)

    for j in range(CHUNKS_PER_WORKER):
        c = wid + NUM_WORKERS * j

        @pl.when(c < NCHUNKS)
        def _():
            pltpu.async_copy(idx_hbm.at[pl.ds(c * CHUNK, CHUNK)],
                             idx_all.at[pl.ds(j * CHUNK, CHUNK)], sem_i)
    for j in range(CHUNKS_PER_WORKER):
        c = wid + NUM_WORKERS * j

        @pl.when(c < NCHUNKS)
        def _():
            pltpu.make_async_copy(idx_hbm.at[pl.ds(c * CHUNK, CHUNK)],
                                  idx_all.at[pl.ds(j * CHUNK, CHUNK)],
                                  sem_i).wait()

    plsc.subcore_barrier()

    for j in range(CHUNKS_PER_WORKER):
        c = wid + NUM_WORKERS * j
        rows = rows0 if j % 2 == 0 else rows1
        sem_s = sem_s0 if j % 2 == 0 else sem_s1

        @pl.when(c < NCHUNKS)
        def _():
            if j >= 2:
                pltpu.make_async_copy(
                    rows, out_hbm.at[pl.ds(c * CHUNK, CHUNK)], sem_s).wait()
            pltpu.async_copy(
                table_sh.at[idx_all.at[pl.ds(j * CHUNK, CHUNK)]],
                rows, sem_g).wait()
            pltpu.async_copy(rows, out_hbm.at[pl.ds(c * CHUNK, CHUNK)], sem_s)

    pltpu.make_async_copy(rows0, out_hbm.at[pl.ds(0, CHUNK)], sem_s0).wait()
    pltpu.make_async_copy(rows1, out_hbm.at[pl.ds(0, CHUNK)], sem_s1).wait()


TC_BLOCK = 5000
TC_NBLOCKS = NUM_NODES // TC_BLOCK


def _tc_onehot_body(idx_ref, table_ref, out_ref):
    ids = idx_ref[0, 0, :]
    onehot = (ids[:, None] == lax.broadcasted_iota(
        jnp.int32, (TC_BLOCK, EMB_DIM), 1)).astype(jnp.float32)
    out_ref[...] = jnp.dot(onehot, table_ref[...],
                           preferred_element_type=jnp.float32)


def _tc_gather(idx, emb):
    table_p = jnp.zeros((EMB_DIM, EMB_DIM), jnp.float32).at[:NUM_TYPES].set(emb)
    idx3 = idx.reshape(TC_NBLOCKS, 1, TC_BLOCK)
    return pl.pallas_call(
        _tc_onehot_body,
        grid=(TC_NBLOCKS,),
        in_specs=[
            pl.BlockSpec((1, 1, TC_BLOCK), lambda i: (i, 0, 0)),
            pl.BlockSpec((EMB_DIM, EMB_DIM), lambda i: (0, 0)),
        ],
        out_specs=pl.BlockSpec((TC_BLOCK, EMB_DIM), lambda i: (i, 0)),
        out_shape=jax.ShapeDtypeStruct((NUM_NODES, EMB_DIM), jnp.float32),
    )(idx3, table_p)


def kernel(atomic_numbers, emb):
    idx = atomic_numbers.astype(jnp.int32)
    mesh = plsc.VectorSubcoreMesh(
        core_axis_name="c", subcore_axis_name="s",
        num_cores=NUM_CORES, num_subcores=NUM_SUBCORES)
    out_sc = pl.kernel(
        _emb_lookup_body,
        out_type=jax.ShapeDtypeStruct((NUM_NODES, EMB_DIM), jnp.float32),
        mesh=mesh,
        scratch_types=[
            pltpu.VMEM_SHARED((NUM_TYPES, EMB_DIM), jnp.float32),
            pltpu.VMEM((CHUNK * CHUNKS_PER_WORKER,), jnp.int32),
            pltpu.VMEM((CHUNK, EMB_DIM), jnp.float32),
            pltpu.VMEM((CHUNK, EMB_DIM), jnp.float32),
            pltpu.SemaphoreType.DMA,
            pltpu.SemaphoreType.DMA,
            pltpu.SemaphoreType.DMA,
            pltpu.SemaphoreType.DMA,
        ],
    )(emb, idx)
    out_tc = _tc_gather(idx, emb)
    return (out_sc, out_tc)
